# speedup vs baseline: 26.7124x; 26.7124x over previous
import jax
import jax.numpy as jnp
from jax import lax
from jax.experimental import pallas as pl
from jax.experimental.pallas import tpu as pltpu

_B, _H, _W, _NR = 4, 224, 224, 14
_BS = _H // _NR
_S = _NR * _NR
_ROWS = _B * _S
_Q = _B * _NR
_NPIX = _H * _W


def _iota2(shape, dim):
    return lax.broadcasted_iota(jnp.int32, shape, dim)


def _feature_kernel(x_ref, seg_ref, feat_ref):
    f32 = jnp.float32

    yi = _iota2((_H, _W), 0)
    xi = _iota2((_H, _W), 1)
    seg2 = (yi // _BS) * _NR + xi // _BS
    seg_ref[...] = jnp.broadcast_to(seg2[None], (_B, _H, _W))

    PT = (_iota2((_W, _NR), 0) // _BS == _iota2((_W, _NR), 1)).astype(f32)
    P2 = (_iota2((_Q, _B * _H), 1) // _BS == _iota2((_Q, _B * _H), 0)).astype(f32)
    qi = _iota2((_Q, _B * _H), 0)
    R = (_iota2((_Q, _B * _H), 1) == (qi // _NR) * _H + (qi % _NR) * _BS + 8).astype(f32)
    CT = (_iota2((_W, _NR), 0) == _iota2((_W, _NR), 1) * _BS + 8).astype(f32)

    def mm(a, b):
        return jnp.dot(a, b, preferred_element_type=f32)

    x0 = x_ref[:, 0].reshape(_B * _H, _W)
    colsum = mm(x0, PT)
    sv_grid = mm(P2, colsum)
    sv2_grid = mm(P2, mm(x0 * x0, PT))

    rgb_grids = []
    for c in range(3):
        xc = x_ref[:, c].reshape(_B * _H, _W)
        rgb_grids.append(mm(mm(R, xc), CT))

    OneQ = (_iota2((_ROWS, _Q), 0) // _NR == _iota2((_ROWS, _Q), 1)).astype(f32)
    OneJ = (_iota2((_ROWS, _NR), 0) % _NR == _iota2((_ROWS, _NR), 1)).astype(f32)

    def to_col(grid):
        return jnp.sum(mm(OneQ, grid) * OneJ, axis=1, keepdims=True)

    sv = to_col(sv_grid)
    sv2 = to_col(sv2_grid)
    rcol = to_col(rgb_grids[0])
    gcol = to_col(rgb_grids[1])
    bcol = to_col(rgb_grids[2])

    mean = sv * (1.0 / _NPIX)
    var = jnp.maximum(sv2 - _NPIX * mean * mean, 0.0) * (1.0 / (_NPIX - 1))
    std = jnp.sqrt(var + 1e-12)

    ridx = _iota2((_ROWS, 1), 0)
    xcen = ((ridx % _NR) * _BS).astype(f32) + 7.5
    ycen = (((ridx // _NR) % _NR) * _BS).astype(f32) + 7.5

    cols = [xcen, ycen, mean, mean, mean, std, std, std, rcol, gcol, bcol]
    lane = _iota2((_ROWS, 16), 1)
    feat = jnp.zeros((_ROWS, 16), f32)
    for k, col in enumerate(cols):
        feat = feat + col * (lane == k).astype(f32)
    feat_ref[...] = feat


def _mlp_kernel(feat_ref,
                W1_ref, b1_ref, g1_ref, be1_ref,
                W2_ref, b2_ref, g2_ref, be2_ref,
                W3_ref, b3_ref, g3_ref, be3_ref,
                W4_ref, b4_ref, g4_ref, be4_ref,
                W5_ref, b5_ref, g5_ref, be5_ref,
                y_ref):
    f32 = jnp.float32
    X = feat_ref[...]
    layers = [
        (W1_ref, b1_ref, g1_ref, be1_ref, True),
        (W2_ref, b2_ref, g2_ref, be2_ref, True),
        (W3_ref, b3_ref, g3_ref, be3_ref, True),
        (W4_ref, b4_ref, g4_ref, be4_ref, True),
        (W5_ref, b5_ref, g5_ref, be5_ref, False),
    ]
    for Wr, br, gr, ber, relu in layers:
        Wt = Wr[...]
        y = jnp.dot(X, Wt, preferred_element_type=f32) + br[...]
        mu = jnp.mean(y, axis=0, keepdims=True)
        d = y - mu
        var = jnp.mean(d * d, axis=0, keepdims=True)
        y = d * (gr[...] / jnp.sqrt(var + 1e-5)) + ber[...]
        if relu:
            y = jnp.maximum(y, 0.0)
        X = y
    y_ref[...] = X


def kernel(x, org_x, W1, b1, g1, be1, W2, b2, g2, be2, W3, b3, g3, be3,
           W4, b4, g4, be4, W5, b5, g5, be5):
    del org_x
    f32 = jnp.float32

    seg, feat = pl.pallas_call(
        _feature_kernel,
        out_shape=(
            jax.ShapeDtypeStruct((_B, _H, _W), jnp.int32),
            jax.ShapeDtypeStruct((_ROWS, 16), f32),
        ),
    )(x)

    W1t = jnp.zeros((16, W1.shape[0]), f32).at[:W1.shape[1]].set(W1.T)
    args = [feat,
            W1t, b1.reshape(1, -1), g1.reshape(1, -1), be1.reshape(1, -1),
            W2.T, b2.reshape(1, -1), g2.reshape(1, -1), be2.reshape(1, -1),
            W3.T, b3.reshape(1, -1), g3.reshape(1, -1), be3.reshape(1, -1),
            W4.T, b4.reshape(1, -1), g4.reshape(1, -1), be4.reshape(1, -1),
            W5.T, b5.reshape(1, -1), g5.reshape(1, -1), be5.reshape(1, -1)]

    ymat = pl.pallas_call(
        _mlp_kernel,
        out_shape=jax.ShapeDtypeStruct((_ROWS, W5.shape[0]), f32),
    )(*args)

    y = ymat.reshape(_B, _S, -1).transpose(0, 2, 1).reshape(_B, -1, _NR, _NR)
    return (seg, y)


# speedup vs baseline: 31.6660x; 1.1854x over previous
import jax
import jax.numpy as jnp
from jax import lax
from jax.experimental import pallas as pl
from jax.experimental.pallas import tpu as pltpu

_B, _H, _W, _NR = 4, 224, 224, 14
_BS = _H // _NR
_S = _NR * _NR
_ROWS = _B * _S
_Q = _B * _NR
_NPIX = _H * _W


def _iota2(shape, dim):
    return lax.broadcasted_iota(jnp.int32, shape, dim)


def _fused_kernel(x_ref,
                  W1_ref, b1_ref, g1_ref, be1_ref,
                  W2_ref, b2_ref, g2_ref, be2_ref,
                  W3_ref, b3_ref, g3_ref, be3_ref,
                  W4_ref, b4_ref, g4_ref, be4_ref,
                  W5_ref, b5_ref, g5_ref, be5_ref,
                  seg_ref, y_ref):
    f32 = jnp.float32

    yi = _iota2((_H, _W), 0)
    xi = _iota2((_H, _W), 1)
    seg2 = (yi // _BS) * _NR + xi // _BS
    seg_ref[...] = jnp.broadcast_to(seg2[None], (_B, _H, _W))

    PT = (_iota2((_W, _NR), 0) // _BS == _iota2((_W, _NR), 1)).astype(f32)
    P2 = (_iota2((_Q, _B * _H), 1) // _BS == _iota2((_Q, _B * _H), 0)).astype(f32)
    qi = _iota2((_Q, _B * _H), 0)
    R = (_iota2((_Q, _B * _H), 1) == (qi // _NR) * _H + (qi % _NR) * _BS + 8).astype(f32)
    CT = (_iota2((_W, _NR), 0) == _iota2((_W, _NR), 1) * _BS + 8).astype(f32)

    def mm(a, b):
        return jnp.dot(a, b, preferred_element_type=f32)

    x0 = x_ref[:, 0].reshape(_B * _H, _W)
    colsum = mm(x0, PT)
    sv_grid = mm(P2, colsum)
    sv2_grid = mm(P2, mm(x0 * x0, PT))

    rgb_grids = []
    for c in range(3):
        xc = x_ref[:, c].reshape(_B * _H, _W)
        rgb_grids.append(mm(mm(R, xc), CT))

    OneQ = (_iota2((_ROWS, _Q), 0) // _NR == _iota2((_ROWS, _Q), 1)).astype(f32)
    OneJ = (_iota2((_ROWS, _NR), 0) % _NR == _iota2((_ROWS, _NR), 1)).astype(f32)

    def to_col(grid):
        return jnp.sum(mm(OneQ, grid) * OneJ, axis=1, keepdims=True)

    sv = to_col(sv_grid)
    sv2 = to_col(sv2_grid)
    rcol = to_col(rgb_grids[0])
    gcol = to_col(rgb_grids[1])
    bcol = to_col(rgb_grids[2])

    mean = sv * (1.0 / _NPIX)
    var = jnp.maximum(sv2 - _NPIX * mean * mean, 0.0) * (1.0 / (_NPIX - 1))
    std = jnp.sqrt(var + 1e-12)

    ridx = _iota2((_ROWS, 1), 0)
    xcen = ((ridx % _NR) * _BS).astype(f32) + 7.5
    ycen = (((ridx // _NR) % _NR) * _BS).astype(f32) + 7.5

    cols = [xcen, ycen, mean, mean, mean, std, std, std, rcol, gcol, bcol]
    lane = _iota2((_ROWS, 16), 1)
    X = jnp.zeros((_ROWS, 16), f32)
    for k, col in enumerate(cols):
        X = X + col * (lane == k).astype(f32)

    layers = [
        (W1_ref, b1_ref, g1_ref, be1_ref, True),
        (W2_ref, b2_ref, g2_ref, be2_ref, True),
        (W3_ref, b3_ref, g3_ref, be3_ref, True),
        (W4_ref, b4_ref, g4_ref, be4_ref, True),
        (W5_ref, b5_ref, g5_ref, be5_ref, False),
    ]
    for li, (Wr, br, gr, ber, relu) in enumerate(layers):
        Wt = Wr[...]
        Xin = X if li > 0 else X[:, :W1_ref.shape[1]]
        y = lax.dot_general(Xin, Wt, (((1,), (1,)), ((), ())),
                            preferred_element_type=f32) + br[...]
        mu = jnp.mean(y, axis=0, keepdims=True)
        d = y - mu
        var = jnp.mean(d * d, axis=0, keepdims=True)
        y = d * (gr[...] / jnp.sqrt(var + 1e-5)) + ber[...]
        if relu:
            y = jnp.maximum(y, 0.0)
        X = y
    y_ref[...] = X


def kernel(x, org_x, W1, b1, g1, be1, W2, b2, g2, be2, W3, b3, g3, be3,
           W4, b4, g4, be4, W5, b5, g5, be5):
    del org_x
    f32 = jnp.float32

    args = [x,
            W1, b1.reshape(1, -1), g1.reshape(1, -1), be1.reshape(1, -1),
            W2, b2.reshape(1, -1), g2.reshape(1, -1), be2.reshape(1, -1),
            W3, b3.reshape(1, -1), g3.reshape(1, -1), be3.reshape(1, -1),
            W4, b4.reshape(1, -1), g4.reshape(1, -1), be4.reshape(1, -1),
            W5, b5.reshape(1, -1), g5.reshape(1, -1), be5.reshape(1, -1)]

    seg, ymat = pl.pallas_call(
        _fused_kernel,
        out_shape=(
            jax.ShapeDtypeStruct((_B, _H, _W), jnp.int32),
            jax.ShapeDtypeStruct((_ROWS, W5.shape[0]), f32),
        ),
    )(*args)

    y = ymat.reshape(_B, _S, -1).transpose(0, 2, 1).reshape(_B, -1, _NR, _NR)
    return (seg, y)


# speedup vs baseline: 41.7106x; 1.3172x over previous
import jax
import jax.numpy as jnp
from jax import lax
from jax.experimental import pallas as pl
from jax.experimental.pallas import tpu as pltpu

_B, _H, _W, _NR = 4, 224, 224, 14
_BS = _H // _NR
_S = _NR * _NR
_ROWS = _B * _S
_Q = _B * _NR
_NPIX = _H * _W
_CH = [11, 96, 192, 384, 768, 768]
_VOFF = [0, 96, 288, 672, 1440]
_VTOT = 2208


def _iota(shape, dim):
    return lax.broadcasted_iota(jnp.int32, shape, dim)


def _fused_kernel(x_hbm, W1_hbm, W2_hbm, W3_hbm, W4_hbm, W5_hbm, vec_hbm,
                  seg_hbm, y_hbm,
                  x_v, w1_v, w2_v, w3_v, w4_v, w5_v, vec_v, seg_v, y_v,
                  sems):
    f32 = jnp.float32

    cp_x = pltpu.make_async_copy(x_hbm, x_v, sems.at[0])
    cp_w = [pltpu.make_async_copy(src, dst, sems.at[1 + i])
            for i, (src, dst) in enumerate(
                [(W1_hbm, w1_v), (W2_hbm, w2_v), (W3_hbm, w3_v),
                 (W4_hbm, w4_v), (W5_hbm, w5_v)])]
    cp_vec = pltpu.make_async_copy(vec_hbm, vec_v, sems.at[6])
    cp_x.start()
    cp_vec.start()
    for c in cp_w:
        c.start()

    yi = _iota((_H, _W), 0)
    xi = _iota((_H, _W), 1)
    seg2 = (yi // _BS) * _NR + xi // _BS
    seg_v[...] = jnp.broadcast_to(seg2[None], (_B, _H, _W))
    cp_seg = pltpu.make_async_copy(seg_v, seg_hbm, sems.at[7])
    cp_seg.start()

    PT = (_iota((_W, _NR), 0) // _BS == _iota((_W, _NR), 1)).astype(f32)
    P2 = (_iota((_Q, _B * _H), 1) // _BS == _iota((_Q, _B * _H), 0)).astype(f32)
    R = (_iota((_B, _NR, _B * _H), 2)
         == _iota((_B, _NR, _B * _H), 0) * _H
         + _iota((_B, _NR, _B * _H), 1) * _BS + 8).astype(f32)
    R = R.reshape(_Q, _B * _H)
    CT = (_iota((_W, _NR), 0) == _iota((_W, _NR), 1) * _BS + 8).astype(f32)

    def mm(a, b):
        return jnp.dot(a, b, preferred_element_type=f32)

    cp_x.wait()

    x0 = x_v[:, 0].reshape(_B * _H, _W)
    colsum = mm(x0, PT)
    sv_grid = mm(P2, colsum)
    sv2_grid = mm(P2, mm(x0 * x0, PT))

    rgb_grids = []
    for c in range(3):
        xc = x_v[:, c].reshape(_B * _H, _W)
        rgb_grids.append(mm(mm(R, xc), CT))

    OneQ = (_iota((_Q, _NR, _Q), 0) == _iota((_Q, _NR, _Q), 2)
            ).astype(f32).reshape(_ROWS, _Q)
    OneJ = (_iota((_Q, _NR, _NR), 1) == _iota((_Q, _NR, _NR), 2)
            ).astype(f32).reshape(_ROWS, _NR)

    def to_col(grid):
        return jnp.sum(mm(OneQ, grid) * OneJ, axis=1, keepdims=True)

    sv = to_col(sv_grid)
    sv2 = to_col(sv2_grid)
    rcol = to_col(rgb_grids[0])
    gcol = to_col(rgb_grids[1])
    bcol = to_col(rgb_grids[2])

    mean = sv * (1.0 / _NPIX)
    var = jnp.maximum(sv2 - _NPIX * mean * mean, 0.0) * (1.0 / (_NPIX - 1))
    std = jnp.sqrt(var + 1e-12)

    xcen = (_iota((_Q, _NR, 1), 1) * _BS).astype(f32).reshape(_ROWS, 1) + 7.5
    ycen = ((_iota((_B, _NR, _NR, 1), 1) * _BS).astype(f32)
            .reshape(_ROWS, 1) + 7.5)

    cols = [xcen, ycen, mean, mean, mean, std, std, std, rcol, gcol, bcol]
    lane = _iota((_ROWS, 16), 1)
    X = jnp.zeros((_ROWS, 16), f32)
    for k, col in enumerate(cols):
        X = X + col * (lane == k).astype(f32)
    X = X[:, :_CH[0]]

    cp_vec.wait()
    vecs = vec_v[...]
    for li, wv in enumerate([w1_v, w2_v, w3_v, w4_v, w5_v]):
        cp_w[li].wait()
        Wt = wv[...]
        cout, off = _CH[li + 1], _VOFF[li]
        br = vecs[0:1, off:off + cout]
        gr = vecs[1:2, off:off + cout]
        ber = vecs[2:3, off:off + cout]
        y = lax.dot_general(X, Wt, (((1,), (1,)), ((), ())),
                            preferred_element_type=f32) + br
        mu = jnp.mean(y, axis=0, keepdims=True)
        d = y - mu
        var = jnp.mean(d * d, axis=0, keepdims=True)
        y = d * (gr / jnp.sqrt(var + 1e-5)) + ber
        if li < 4:
            y = jnp.maximum(y, 0.0)
        X = y
    y_v[...] = X
    cp_y = pltpu.make_async_copy(y_v, y_hbm, sems.at[8])
    cp_y.start()
    cp_y.wait()
    cp_seg.wait()


def kernel(x, org_x, W1, b1, g1, be1, W2, b2, g2, be2, W3, b3, g3, be3,
           W4, b4, g4, be4, W5, b5, g5, be5):
    del org_x
    f32 = jnp.float32

    vecs = jnp.stack([jnp.concatenate([b1, b2, b3, b4, b5]),
                      jnp.concatenate([g1, g2, g3, g4, g5]),
                      jnp.concatenate([be1, be2, be3, be4, be5])])

    any_spec = pl.BlockSpec(memory_space=pl.ANY)
    seg, ymat = pl.pallas_call(
        _fused_kernel,
        in_specs=[any_spec] * 7,
        out_specs=(any_spec, any_spec),
        out_shape=(
            jax.ShapeDtypeStruct((_B, _H, _W), jnp.int32),
            jax.ShapeDtypeStruct((_ROWS, _CH[5]), f32),
        ),
        scratch_shapes=[
            pltpu.VMEM((_B, 3, _H, _W), f32),
            pltpu.VMEM((_CH[1], _CH[0]), f32),
            pltpu.VMEM((_CH[2], _CH[1]), f32),
            pltpu.VMEM((_CH[3], _CH[2]), f32),
            pltpu.VMEM((_CH[4], _CH[3]), f32),
            pltpu.VMEM((_CH[5], _CH[4]), f32),
            pltpu.VMEM((3, _VTOT), f32),
            pltpu.VMEM((_B, _H, _W), jnp.int32),
            pltpu.VMEM((_ROWS, _CH[5]), f32),
            pltpu.SemaphoreType.DMA((9,)),
        ],
    )(x, W1, W2, W3, W4, W5, vecs)

    y = ymat.reshape(_B, _S, -1).transpose(0, 2, 1).reshape(_B, -1, _NR, _NR)
    return (seg, y)
